# Initial kernel scaffold; baseline (speedup 1.0000x reference)
import jax
import jax.numpy as jnp
from jax.experimental import pallas as pl


def kernel(x, edge_index, batch, pre_w1, pre_b1, pre_w2, pre_b2, conv_w0, conv_b0, conv_w1, conv_b1, f0_w1, f0_b1, f0_w2, f0_b2, f1_w1, f1_b1, f1_w2, f1_b2, bn_g0, bn_b0, bn_g1, bn_b1, post_w1, post_b1, post_w2, post_b2):
    raise NotImplementedError("write your pallas kernel here")


# speedup vs baseline: 7.6098x; 7.6098x over previous
import functools

import jax
import jax.numpy as jnp
from jax import lax
from jax.experimental import pallas as pl
from jax.experimental.pallas import tpu as pltpu
from jax.experimental.pallas import tpu_sc as plsc

N = 10000
E = 320000
D = 128
G = 128

NC = 2
NS = 16
NW = NC * NS
L = 16

CHUNK = 128
CPW = 80
EPW = CHUNK * CPW
E_PAD = NW * EPW
ZROW = N
TILE_ROWS = 640
ACC_ROWS = NS * TILE_ROWS
EPW_DEG = E // NW
SEG = ACC_ROWS // NS

_sc_mesh = plsc.VectorSubcoreMesh(core_axis_name="c", subcore_axis_name="s",
                                  num_cores=NC, num_subcores=NS)


@functools.partial(
    pl.kernel,
    out_type=jax.ShapeDtypeStruct((NC, ACC_ROWS, D), jnp.float32),
    mesh=_sc_mesh,
    scratch_types=[
        pltpu.VMEM((EPW_DEG,), jnp.int32),
        pltpu.VMEM((ACC_ROWS,), jnp.float32),
        pltpu.VMEM((NS, SEG), jnp.float32),
        pltpu.VMEM((SEG,), jnp.float32),
        pltpu.VMEM((SEG, D), jnp.float32),
        pltpu.VMEM_SHARED((NS, ACC_ROWS), jnp.float32),
    ],
    compiler_params=pltpu.CompilerParams(needs_layout_passes=False),
)
def _deg_kernel(dst_hbm, out_hbm, dst_v, hist_v, seg_v, tot_v, bc_v, part_sh):
    c = lax.axis_index("c")
    s = lax.axis_index("s")
    w = c * NS + s
    pltpu.sync_copy(dst_hbm.at[w], dst_v)

    zero16 = jnp.zeros((L,), jnp.float32)

    def zbody(k, carry):
        hist_v[pl.ds(k * L, L)] = zero16
        return carry

    lax.fori_loop(0, ACC_ROWS // L, zbody, None, unroll=8)

    ones16 = jnp.ones((L,), jnp.float32)

    def body(i, carry):
        idx = dst_v[pl.ds(i * L, L)]
        plsc.addupdate_scatter(hist_v, [idx], ones16)
        return carry

    lax.fori_loop(0, EPW_DEG // L, body, None, unroll=8)

    pltpu.sync_copy(hist_v, part_sh.at[s])
    plsc.subcore_barrier()
    pltpu.sync_copy(part_sh.at[:, pl.ds(s * SEG, SEG)], seg_v)

    def red(k, carry):
        acc = seg_v[0, pl.ds(k * L, L)]
        for r in range(1, NS):
            acc = acc + seg_v[r, pl.ds(k * L, L)]
        tot_v[pl.ds(k * L, L)] = acc
        return carry

    lax.fori_loop(0, SEG // L, red, None, unroll=2)

    def bc(k, carry):
        t16 = tot_v[pl.ds(k * L, L)]
        for l in range(L):
            v = jnp.full((L,), t16[l], jnp.float32)
            for j in range(D // L):
                bc_v[k * L + l, pl.ds(j * L, L)] = v
        return carry

    lax.fori_loop(0, SEG // L, bc, None)
    pltpu.sync_copy(bc_v, out_hbm.at[c, pl.ds(s * SEG, SEG)])


@functools.partial(
    pl.kernel,
    out_type=jax.ShapeDtypeStruct((NC, ACC_ROWS, D), jnp.float32),
    mesh=_sc_mesh,
    scratch_types=[
        pltpu.VMEM((CPW, CHUNK), jnp.int32),
        pltpu.VMEM((CPW, CHUNK), jnp.int32),
        pltpu.VMEM((CHUNK, D), jnp.float32),
        pltpu.VMEM_SHARED((ACC_ROWS, D), jnp.float32),
        pltpu.SemaphoreType.DMA,
    ],
    compiler_params=pltpu.CompilerParams(needs_layout_passes=False),
)
def _conv_kernel(m_hbm, src_hbm, dst_hbm, out_hbm,
                 src_v, dst_v, rows_v, acc_sh, gsem):
    c = lax.axis_index("c")
    s = lax.axis_index("s")
    w = c * NS + s
    pltpu.sync_copy(src_hbm.at[w], src_v)
    pltpu.sync_copy(dst_hbm.at[w], dst_v)

    zero16 = jnp.zeros((L,), jnp.float32)

    def zb(k, carry):
        rows_v[k // (D // L), pl.ds((k % (D // L)) * L, L)] = zero16
        return carry

    lax.fori_loop(0, CHUNK * (D // L), zb, None, unroll=8)

    def zacc(k, carry):
        pltpu.sync_copy(
            rows_v, acc_sh.at[pl.ds(s * TILE_ROWS + k * CHUNK, CHUNK)])
        return carry

    lax.fori_loop(0, TILE_ROWS // CHUNK, zacc, None)
    plsc.subcore_barrier()

    def body(j, carry):
        pltpu.async_copy(m_hbm.at[src_v.at[j]], rows_v, gsem).wait()
        pltpu.sync_copy(rows_v, acc_sh.at[dst_v.at[j]], add=True)
        return carry

    lax.fori_loop(0, CPW, body, None)
    plsc.subcore_barrier()
    pltpu.sync_copy(acc_sh.at[pl.ds(s * TILE_ROWS, TILE_ROWS)],
                    out_hbm.at[c, pl.ds(s * TILE_ROWS, TILE_ROWS)])


def _gelu(x):
    return x * 0.5 * (1.0 + lax.erf(x * 0.7071067811865476))


def _mm(a, b):
    return jnp.dot(a, b, preferred_element_type=jnp.float32,
                   precision=lax.Precision.HIGHEST)


def _dinv(degp):
    return lax.rsqrt(degp[0, :N] + degp[1, :N] + 1.0)


def _p1_body(x_ref, degp_ref, w1_ref, b1_ref, w2_ref, b2_ref, cw0_ref,
             h0_ref, m0_ref, dinv_ref):
    dinv = _dinv(degp_ref[...])
    h = _gelu(_mm(x_ref[...], w1_ref[...]) + b1_ref[...])
    h = _mm(h, w2_ref[...]) + b2_ref[...]
    h0_ref[...] = h
    m0_ref[...] = _mm(h, cw0_ref[...]) * dinv
    dinv_ref[...] = dinv


def _comb_body(part_ref, m_ref, hres_ref, dinv_ref, cb_ref,
               t_ref, mu_ref, var_ref):
    seg = part_ref[0, :N] + part_ref[1, :N]
    t = (seg + m_ref[...]) * dinv_ref[...] + cb_ref[...] + hres_ref[...]
    mean = jnp.mean(t, axis=0, keepdims=True)
    var = jnp.mean((t - mean) ** 2, axis=0, keepdims=True)
    t_ref[...] = t
    mu_ref[...] = mean
    var_ref[...] = var


def _ffnn_body(t_ref, mu_ref, var_ref, g_ref, bb_ref,
               fw1_ref, fb1_ref, fw2_ref, fb2_ref, cw_ref, dinv_ref,
               v_ref, m_ref):
    tb = ((t_ref[...] - mu_ref[...]) * lax.rsqrt(var_ref[...] + 1e-5)
          * g_ref[...] + bb_ref[...])
    u = _mm(_gelu(_mm(tb, fw1_ref[...]) + fb1_ref[...]), fw2_ref[...])
    u = u + fb2_ref[...] + tb
    v_ref[...] = u
    m_ref[...] = _mm(u, cw_ref[...]) * dinv_ref[...]


def _final_body(t_ref, mu_ref, var_ref, g1_ref, bb1_ref,
                fw1_ref, fb1_ref, fw2_ref, fb2_ref, batch_ref,
                pw1_ref, pb1_ref, pw2_ref, pb2_ref, out_ref):
    tb = ((t_ref[...] - mu_ref[...]) * lax.rsqrt(var_ref[...] + 1e-5)
          * g1_ref[...] + bb1_ref[...])
    z = _mm(_gelu(_mm(tb, fw1_ref[...]) + fb1_ref[...]), fw2_ref[...])
    z = z + fb2_ref[...] + tb
    bt = batch_ref[...]
    iota = lax.broadcasted_iota(jnp.int32, (G, N), 0)
    onehot = (bt[None, :] == iota).astype(jnp.float32)
    sums = _mm(onehot, z)
    cnt = lax.dot_general(onehot, jnp.ones((N, 1), jnp.float32),
                          (((1,), (0,)), ((), ())),
                          preferred_element_type=jnp.float32,
                          precision=lax.Precision.HIGHEST)
    pooled = sums / jnp.maximum(cnt, 1.0)
    out = _mm(_gelu(_mm(pooled, pw1_ref[...]) + pb1_ref[...]), pw2_ref[...])
    out_ref[...] = out + pb2_ref[...]


def _f32(shape):
    return jax.ShapeDtypeStruct(shape, jnp.float32)


_p1 = pl.pallas_call(
    _p1_body, out_shape=[_f32((N, D)), _f32((N, D)), _f32((N, D))])
_comb = pl.pallas_call(
    _comb_body, out_shape=[_f32((N, D)), _f32((1, D)), _f32((1, D))])
_ffnn = pl.pallas_call(_ffnn_body, out_shape=[_f32((N, D)), _f32((N, D))])
_final = pl.pallas_call(_final_body, out_shape=_f32((G, D)))


def kernel(x, edge_index, batch, pre_w1, pre_b1, pre_w2, pre_b2, conv_w0,
           conv_b0, conv_w1, conv_b1, f0_w1, f0_b1, f0_w2, f0_b2, f1_w1,
           f1_b1, f1_w2, f1_b2, bn_g0, bn_b0, bn_g1, bn_b1, post_w1,
           post_b1, post_w2, post_b2):
    src = edge_index[0]
    dst = edge_index[1]
    pad = E_PAD - E
    src_p = jnp.concatenate(
        [src, jnp.full((pad,), ZROW, src.dtype)]).reshape(NW, CPW, CHUNK)
    dst_p = jnp.concatenate(
        [dst, jnp.zeros((pad,), dst.dtype)]).reshape(NW, CPW, CHUNK)

    degp = _deg_kernel(dst.reshape(NW, EPW_DEG))
    h0, m0, dinv = _p1(x, degp, pre_w1, pre_b1, pre_w2, pre_b2, conv_w0)
    zrows = jnp.zeros((8, D), jnp.float32)
    part0 = _conv_kernel(jnp.concatenate([m0, zrows]), src_p, dst_p)
    t0, mu0, var0 = _comb(part0, m0, h0, dinv, conv_b0)
    v, m1 = _ffnn(t0, mu0, var0, bn_g0, bn_b0,
                  f0_w1, f0_b1, f0_w2, f0_b2, conv_w1, dinv)
    part1 = _conv_kernel(jnp.concatenate([m1, zrows]), src_p, dst_p)
    t1, mu1, var1 = _comb(part1, m1, v, dinv, conv_b1)
    out = _final(t1, mu1, var1, bn_g1, bn_b1,
                 f1_w1, f1_b1, f1_w2, f1_b2, batch,
                 post_w1, post_b1, post_w2, post_b2)
    return out


# speedup vs baseline: 8.2051x; 1.0782x over previous
import functools

import jax
import jax.numpy as jnp
from jax import lax
from jax.experimental import pallas as pl
from jax.experimental.pallas import tpu as pltpu
from jax.experimental.pallas import tpu_sc as plsc

N = 10000
E = 320000
D = 128
G = 128

NC = 2
NS = 16
NW = NC * NS
L = 16

CHUNK = 128
CPW = 80
EPW = CHUNK * CPW
E_PAD = NW * EPW
ZROW = N
TILE_ROWS = 640
ACC_ROWS = NS * TILE_ROWS
EPW_DEG = E // NW
SEG = ACC_ROWS // NS

_sc_mesh = plsc.VectorSubcoreMesh(core_axis_name="c", subcore_axis_name="s",
                                  num_cores=NC, num_subcores=NS)


@functools.partial(
    pl.kernel,
    out_type=jax.ShapeDtypeStruct((NC, ACC_ROWS, D), jnp.float32),
    mesh=_sc_mesh,
    scratch_types=[
        pltpu.VMEM((EPW_DEG,), jnp.int32),
        pltpu.VMEM((ACC_ROWS,), jnp.float32),
        pltpu.VMEM((NS, SEG), jnp.float32),
        pltpu.VMEM((SEG,), jnp.float32),
        pltpu.VMEM((SEG, D), jnp.float32),
        pltpu.VMEM_SHARED((NS, ACC_ROWS), jnp.float32),
    ],
    compiler_params=pltpu.CompilerParams(needs_layout_passes=False),
)
def _deg_kernel(dst_hbm, out_hbm, dst_v, hist_v, seg_v, tot_v, bc_v, part_sh):
    c = lax.axis_index("c")
    s = lax.axis_index("s")
    w = c * NS + s
    pltpu.sync_copy(dst_hbm.at[w], dst_v)

    zero16 = jnp.zeros((L,), jnp.float32)

    def zbody(k, carry):
        hist_v[pl.ds(k * L, L)] = zero16
        return carry

    lax.fori_loop(0, ACC_ROWS // L, zbody, None, unroll=8)

    ones16 = jnp.ones((L,), jnp.float32)

    def body(i, carry):
        idx = dst_v[pl.ds(i * L, L)]
        plsc.addupdate_scatter(hist_v, [idx], ones16)
        return carry

    lax.fori_loop(0, EPW_DEG // L, body, None, unroll=8)

    pltpu.sync_copy(hist_v, part_sh.at[s])
    plsc.subcore_barrier()
    pltpu.sync_copy(part_sh.at[:, pl.ds(s * SEG, SEG)], seg_v)

    def red(k, carry):
        acc = seg_v[0, pl.ds(k * L, L)]
        for r in range(1, NS):
            acc = acc + seg_v[r, pl.ds(k * L, L)]
        tot_v[pl.ds(k * L, L)] = acc
        return carry

    lax.fori_loop(0, SEG // L, red, None, unroll=2)

    def bc(k, carry):
        t16 = tot_v[pl.ds(k * L, L)]
        for l in range(L):
            v = jnp.full((L,), t16[l], jnp.float32)
            for j in range(D // L):
                bc_v[k * L + l, pl.ds(j * L, L)] = v
        return carry

    lax.fori_loop(0, SEG // L, bc, None)
    pltpu.sync_copy(bc_v, out_hbm.at[c, pl.ds(s * SEG, SEG)])


@functools.partial(
    pl.kernel,
    out_type=jax.ShapeDtypeStruct((NC, ACC_ROWS, D), jnp.float32),
    mesh=_sc_mesh,
    scratch_types=[
        pltpu.VMEM((CPW // 2, CHUNK), jnp.int32),
        pltpu.VMEM((CPW // 2, CHUNK), jnp.int32),
        pltpu.VMEM((CHUNK, D), jnp.float32),
        pltpu.VMEM((CHUNK, D), jnp.float32),
        pltpu.VMEM_SHARED((ACC_ROWS, D), jnp.float32),
        pltpu.SemaphoreType.DMA,
        pltpu.SemaphoreType.DMA,
    ],
    compiler_params=pltpu.CompilerParams(needs_layout_passes=False),
)
def _conv_kernel(m_hbm, src_hbm, dst_hbm, out_hbm,
                 src_v, dst_v, rows0_v, rows1_v, acc_sh, sem0, sem1):
    c = lax.axis_index("c")
    s = lax.axis_index("s")
    w = c * NS + s
    ch = CPW // 2

    zero16 = jnp.zeros((L,), jnp.float32)

    def zb(k, carry):
        rows0_v[k // (D // L), pl.ds((k % (D // L)) * L, L)] = zero16
        return carry

    lax.fori_loop(0, CHUNK * (D // L), zb, None, unroll=8)

    def zacc(k, carry):
        pltpu.sync_copy(
            rows0_v, acc_sh.at[pl.ds(s * TILE_ROWS + k * CHUNK, CHUNK)])
        return carry

    lax.fori_loop(0, TILE_ROWS // CHUNK, zacc, None)
    plsc.subcore_barrier()

    def body(g, carry):
        j = g * 2
        pltpu.async_copy(m_hbm.at[src_v.at[j + 1]], rows1_v, sem1)
        pltpu.make_async_copy(m_hbm.at[src_v.at[j]], rows0_v, sem0).wait()
        pltpu.sync_copy(rows0_v, acc_sh.at[dst_v.at[j]], add=True)
        jn = lax.rem(j + 2, ch)
        pltpu.async_copy(m_hbm.at[src_v.at[jn]], rows0_v, sem0)
        pltpu.make_async_copy(m_hbm.at[src_v.at[j + 1]], rows1_v, sem1).wait()
        pltpu.sync_copy(rows1_v, acc_sh.at[dst_v.at[j + 1]], add=True)
        return carry

    for half in range(2):
        pltpu.sync_copy(src_hbm.at[w, pl.ds(half * ch, ch)], src_v)
        pltpu.sync_copy(dst_hbm.at[w, pl.ds(half * ch, ch)], dst_v)
        pltpu.async_copy(m_hbm.at[src_v.at[0]], rows0_v, sem0)
        lax.fori_loop(0, ch // 2, body, None)
        pltpu.make_async_copy(m_hbm.at[src_v.at[0]], rows0_v, sem0).wait()
    plsc.subcore_barrier()
    pltpu.sync_copy(acc_sh.at[pl.ds(s * TILE_ROWS, TILE_ROWS)],
                    out_hbm.at[c, pl.ds(s * TILE_ROWS, TILE_ROWS)])


def _gelu(x):
    return x * 0.5 * (1.0 + lax.erf(x * 0.7071067811865476))


def _mm(a, b):
    return jnp.dot(a, b, preferred_element_type=jnp.float32,
                   precision=lax.Precision.HIGHEST)


def _dinv(degp):
    return lax.rsqrt(degp[0, :N] + degp[1, :N] + 1.0)


def _p1_body(x_ref, degp_ref, w1_ref, b1_ref, w2_ref, b2_ref, cw0_ref,
             h0_ref, m0_ref, dinv_ref):
    dinv = _dinv(degp_ref[...])
    h = _gelu(_mm(x_ref[...], w1_ref[...]) + b1_ref[...])
    h = _mm(h, w2_ref[...]) + b2_ref[...]
    h0_ref[...] = h
    m0_ref[...] = _mm(h, cw0_ref[...]) * dinv
    dinv_ref[...] = dinv


def _comb_body(part_ref, m_ref, hres_ref, dinv_ref, cb_ref,
               t_ref, mu_ref, var_ref):
    seg = part_ref[0, :N] + part_ref[1, :N]
    t = (seg + m_ref[...]) * dinv_ref[...] + cb_ref[...] + hres_ref[...]
    mean = jnp.mean(t, axis=0, keepdims=True)
    var = jnp.mean((t - mean) ** 2, axis=0, keepdims=True)
    t_ref[...] = t
    mu_ref[...] = mean
    var_ref[...] = var


def _ffnn_body(t_ref, mu_ref, var_ref, g_ref, bb_ref,
               fw1_ref, fb1_ref, fw2_ref, fb2_ref, cw_ref, dinv_ref,
               v_ref, m_ref):
    tb = ((t_ref[...] - mu_ref[...]) * lax.rsqrt(var_ref[...] + 1e-5)
          * g_ref[...] + bb_ref[...])
    u = _mm(_gelu(_mm(tb, fw1_ref[...]) + fb1_ref[...]), fw2_ref[...])
    u = u + fb2_ref[...] + tb
    v_ref[...] = u
    m_ref[...] = _mm(u, cw_ref[...]) * dinv_ref[...]


def _final_body(t_ref, mu_ref, var_ref, g1_ref, bb1_ref,
                fw1_ref, fb1_ref, fw2_ref, fb2_ref, batch_ref,
                pw1_ref, pb1_ref, pw2_ref, pb2_ref, out_ref):
    tb = ((t_ref[...] - mu_ref[...]) * lax.rsqrt(var_ref[...] + 1e-5)
          * g1_ref[...] + bb1_ref[...])
    z = _mm(_gelu(_mm(tb, fw1_ref[...]) + fb1_ref[...]), fw2_ref[...])
    z = z + fb2_ref[...] + tb
    bt = batch_ref[...]
    iota = lax.broadcasted_iota(jnp.int32, (G, N), 0)
    onehot = (bt[None, :] == iota).astype(jnp.float32)
    sums = _mm(onehot, z)
    cnt = lax.dot_general(onehot, jnp.ones((N, 1), jnp.float32),
                          (((1,), (0,)), ((), ())),
                          preferred_element_type=jnp.float32,
                          precision=lax.Precision.HIGHEST)
    pooled = sums / jnp.maximum(cnt, 1.0)
    out = _mm(_gelu(_mm(pooled, pw1_ref[...]) + pb1_ref[...]), pw2_ref[...])
    out_ref[...] = out + pb2_ref[...]


def _f32(shape):
    return jax.ShapeDtypeStruct(shape, jnp.float32)


_p1 = pl.pallas_call(
    _p1_body, out_shape=[_f32((N, D)), _f32((N, D)), _f32((N, D))])
_comb = pl.pallas_call(
    _comb_body, out_shape=[_f32((N, D)), _f32((1, D)), _f32((1, D))])
_ffnn = pl.pallas_call(_ffnn_body, out_shape=[_f32((N, D)), _f32((N, D))])
_final = pl.pallas_call(_final_body, out_shape=_f32((G, D)))


def kernel(x, edge_index, batch, pre_w1, pre_b1, pre_w2, pre_b2, conv_w0,
           conv_b0, conv_w1, conv_b1, f0_w1, f0_b1, f0_w2, f0_b2, f1_w1,
           f1_b1, f1_w2, f1_b2, bn_g0, bn_b0, bn_g1, bn_b1, post_w1,
           post_b1, post_w2, post_b2):
    src = edge_index[0]
    dst = edge_index[1]
    pad = E_PAD - E
    src_p = jnp.concatenate(
        [src, jnp.full((pad,), ZROW, src.dtype)]).reshape(NW, CPW, CHUNK)
    dst_p = jnp.concatenate(
        [dst, jnp.zeros((pad,), dst.dtype)]).reshape(NW, CPW, CHUNK)

    degp = _deg_kernel(dst.reshape(NW, EPW_DEG))
    h0, m0, dinv = _p1(x, degp, pre_w1, pre_b1, pre_w2, pre_b2, conv_w0)
    zrows = jnp.zeros((8, D), jnp.float32)
    part0 = _conv_kernel(jnp.concatenate([m0, zrows]), src_p, dst_p)
    t0, mu0, var0 = _comb(part0, m0, h0, dinv, conv_b0)
    v, m1 = _ffnn(t0, mu0, var0, bn_g0, bn_b0,
                  f0_w1, f0_b1, f0_w2, f0_b2, conv_w1, dinv)
    part1 = _conv_kernel(jnp.concatenate([m1, zrows]), src_p, dst_p)
    t1, mu1, var1 = _comb(part1, m1, v, dinv, conv_b1)
    out = _final(t1, mu1, var1, bn_g1, bn_b1,
                 f1_w1, f1_b1, f1_w2, f1_b2, batch,
                 post_w1, post_b1, post_w2, post_b2)
    return out
